# baseline (speedup 1.0000x reference)
import functools

import jax
import jax.numpy as jnp
from jax import lax
from jax.experimental import pallas as pl
from jax.experimental.pallas import tpu as pltpu
from jax.experimental.pallas import tpu_sc as plsc

VOCAB = 100000
ENC_DIMS = 64
PACK_ROWS = 51200
PRIOR_DIM = 128
BATCH = 16384


_PACK_COLS = 10240


def _pack_body(left_ref, right_ref, o_ref):
    both = jnp.concatenate([left_ref[...], right_ref[...]], axis=0)
    o_ref[...] = both.T


def _pack_base(base_emb):
    bt = base_emb.T
    grid = (PACK_ROWS // _PACK_COLS,)
    shift = PACK_ROWS // _PACK_COLS
    return pl.pallas_call(
        _pack_body,
        grid=grid,
        in_specs=[
            pl.BlockSpec((ENC_DIMS, _PACK_COLS), lambda i: (0, i)),
            pl.BlockSpec(
                (ENC_DIMS, _PACK_COLS),
                lambda i: (0, jnp.minimum(i + shift, (VOCAB - 1) // _PACK_COLS)),
            ),
        ],
        out_specs=pl.BlockSpec((_PACK_COLS, 2 * ENC_DIMS), lambda i: (i, 0)),
        out_shape=jax.ShapeDtypeStruct((PACK_ROWS, 2 * ENC_DIMS), jnp.float32),
    )(bt, bt)


_GATHER_WINDOW = 128
_SC_LANES = 16


def _sc_gather_prior(prior_matrix, idx2):
    n = idx2.shape[0] * idx2.shape[1]

    @functools.partial(
        pl.kernel,
        out_type=jax.ShapeDtypeStruct((n, PRIOR_DIM), jnp.float32),
        mesh=plsc.VectorSubcoreMesh(core_axis_name="core", subcore_axis_name="subcore"),
        compiler_params=pltpu.CompilerParams(
            use_tc_tiling_on_sc=False, has_side_effects=False),
    )
    def gather_kernel(prior_hbm, i_hbm, prior_out_hbm):
        def body(i_vmem, prior_vmem):
            pltpu.sync_copy(
                prior_hbm.at[i_vmem.at[0]],
                prior_vmem.at[pl.ds(0, _GATHER_WINDOW)],
            )
            pltpu.sync_copy(
                prior_hbm.at[i_vmem.at[1]],
                prior_vmem.at[pl.ds(_GATHER_WINDOW, _GATHER_WINDOW)],
            )

        pltpu.emit_pipeline(
            body,
            grid=(n // (2 * _GATHER_WINDOW),),
            in_specs=[pl.BlockSpec((2, _GATHER_WINDOW), index_map=lambda i: (i, 0))],
            out_specs=[
                pl.BlockSpec((2 * _GATHER_WINDOW, PRIOR_DIM), index_map=lambda i: (i, 0)),
            ],
            core_axis_name=("core", "subcore"),
            dimension_semantics=(pltpu.PARALLEL,),
        )(i_hbm, prior_out_hbm)

    return gather_kernel(prior_matrix, idx2)


def _sc_gather_base(packed64, idx2):
    n = idx2.shape[0] * idx2.shape[1]
    mesh = plsc.VectorSubcoreMesh(core_axis_name="core", subcore_axis_name="subcore")

    @functools.partial(
        pl.kernel,
        out_type=jax.ShapeDtypeStruct((n, 2 * ENC_DIMS), jnp.float32),
        mesh=mesh,
        compiler_params=pltpu.CompilerParams(
            use_tc_tiling_on_sc=False, has_side_effects=False),
    )
    def gather_kernel(base_hbm, i_hbm, base_out_hbm):
        def body(i_vmem, base_vmem):
            def scoped(tidx_vmem, rows_vmem, sem):
                @pl.loop(0, _GATHER_WINDOW, step=_SC_LANES)
                def _(k):
                    v = i_vmem[0, pl.ds(k, _SC_LANES)]
                    hi = jnp.where(v >= PACK_ROWS, 1, 0)
                    tidx_vmem[pl.ds(k, _SC_LANES)] = 2 * (v - PACK_ROWS * hi) + hi

                pltpu.async_copy(base_hbm.at[tidx_vmem], rows_vmem, sem).wait()

                @pl.loop(0, _GATHER_WINDOW)
                def _(j):
                    for k in range(ENC_DIMS // _SC_LANES):
                        base_vmem[j, k * _SC_LANES:(k + 1) * _SC_LANES] = (
                            rows_vmem[j, k * _SC_LANES:(k + 1) * _SC_LANES]
                        )

            pl.run_scoped(
                scoped,
                pltpu.VMEM((_GATHER_WINDOW,), jnp.int32),
                pltpu.VMEM((_GATHER_WINDOW, ENC_DIMS), jnp.float32),
                pltpu.SemaphoreType.DMA,
            )

        pltpu.emit_pipeline(
            body,
            grid=(n // _GATHER_WINDOW,),
            in_specs=[pl.BlockSpec((1, _GATHER_WINDOW), index_map=lambda i: (i, 0))],
            out_specs=[
                pl.BlockSpec((_GATHER_WINDOW, 2 * ENC_DIMS), index_map=lambda i: (i, 0)),
            ],
            core_axis_name=("core", "subcore"),
            dimension_semantics=(pltpu.PARALLEL,),
        )(i_hbm, base_out_hbm)

    return gather_kernel(packed64, idx2)


_TC_BLOCK = 4096


def _tc_body(base_ref, prior_ref, w_ref, b_ref, gamma_ref, beta_ref, o_ref):
    prior = jnp.dot(prior_ref[...], w_ref[...], preferred_element_type=jnp.float32)
    h = base_ref[:, :ENC_DIMS] + prior + b_ref[...]
    mu = jnp.mean(h, axis=-1, keepdims=True)
    d = h - mu
    var = jnp.mean(d * d, axis=-1, keepdims=True)
    o_ref[...] = (gamma_ref[...] * (d * lax.rsqrt(var + 1e-5)) + beta_ref[...]).T


def _tc_fuse(base_staged, prior_g, W, b, gamma, beta):
    n = prior_g.shape[0]
    grid = (n // _TC_BLOCK,)
    out_t = pl.pallas_call(
        _tc_body,
        grid=grid,
        in_specs=[
            pl.BlockSpec((_TC_BLOCK, 2 * ENC_DIMS), lambda i: (i, 0)),
            pl.BlockSpec((_TC_BLOCK, PRIOR_DIM), lambda i: (i, 0)),
            pl.BlockSpec((PRIOR_DIM, ENC_DIMS), lambda i: (0, 0)),
            pl.BlockSpec((1, ENC_DIMS), lambda i: (0, 0)),
            pl.BlockSpec((1, ENC_DIMS), lambda i: (0, 0)),
            pl.BlockSpec((1, ENC_DIMS), lambda i: (0, 0)),
        ],
        out_specs=pl.BlockSpec((ENC_DIMS, _TC_BLOCK), lambda i: (0, i)),
        out_shape=jax.ShapeDtypeStruct((ENC_DIMS, n), jnp.float32),
    )(base_staged, prior_g, W, b.reshape(1, -1), gamma.reshape(1, -1), beta.reshape(1, -1))
    return out_t.T


def kernel(base_emb, prior_matrix, W, b, gamma, beta, idx):
    idx2 = idx.reshape(BATCH // _GATHER_WINDOW, _GATHER_WINDOW)
    prior_g = _sc_gather_prior(prior_matrix, idx2)
    packed = _pack_base(base_emb)
    packed64 = packed.reshape(2 * PACK_ROWS, ENC_DIMS)
    base_staged = _sc_gather_base(packed64, idx2)
    return _tc_fuse(base_staged, prior_g, W, b, gamma, beta)


# speedup vs baseline: 1.1166x; 1.1166x over previous
import functools

import jax
import jax.numpy as jnp
from jax import lax
from jax.experimental import pallas as pl
from jax.experimental.pallas import tpu as pltpu
from jax.experimental.pallas import tpu_sc as plsc

VOCAB = 100000
ENC_DIMS = 64
PACK_ROWS = 51200
PRIOR_DIM = 128
BATCH = 16384


_PACK_COLS = 12800


def _pack_body(left_ref, right_ref, o_ref):
    both = jnp.concatenate([left_ref[...], right_ref[...]], axis=0)
    o_ref[...] = both.T


def _pack_base(base_emb):
    bt = base_emb.T
    grid = (PACK_ROWS // _PACK_COLS,)
    shift = PACK_ROWS // _PACK_COLS
    return pl.pallas_call(
        _pack_body,
        grid=grid,
        in_specs=[
            pl.BlockSpec((ENC_DIMS, _PACK_COLS), lambda i: (0, i)),
            pl.BlockSpec(
                (ENC_DIMS, _PACK_COLS),
                lambda i: (0, jnp.minimum(i + shift, (VOCAB - 1) // _PACK_COLS)),
            ),
        ],
        out_specs=pl.BlockSpec((_PACK_COLS, 2 * ENC_DIMS), lambda i: (i, 0)),
        out_shape=jax.ShapeDtypeStruct((PACK_ROWS, 2 * ENC_DIMS), jnp.float32),
    )(bt, bt)


_GATHER_WINDOW = 128
_SC_LANES = 16


def _sc_gather(packed64, prior_matrix, idx2):
    n = idx2.shape[0] * idx2.shape[1]
    mesh = plsc.VectorSubcoreMesh(core_axis_name="core", subcore_axis_name="subcore")

    @functools.partial(
        pl.kernel,
        out_type=(
            jax.ShapeDtypeStruct((n, 2 * ENC_DIMS), jnp.float32),
            jax.ShapeDtypeStruct((n, PRIOR_DIM), jnp.float32),
        ),
        mesh=mesh,
        compiler_params=pltpu.CompilerParams(use_tc_tiling_on_sc=False),
    )
    def gather_kernel(base_hbm, prior_hbm, i_hbm, base_out_hbm, prior_out_hbm):
        def body(i_vmem, base_vmem, prior_vmem):
            def scoped(tidx_vmem, rows_vmem, sem):
                @pl.loop(0, _GATHER_WINDOW, step=_SC_LANES)
                def _(k):
                    v = i_vmem[0, pl.ds(k, _SC_LANES)]
                    hi = jnp.where(v >= PACK_ROWS, 1, 0)
                    tidx_vmem[pl.ds(k, _SC_LANES)] = 2 * (v - PACK_ROWS * hi) + hi

                c0 = pltpu.async_copy(base_hbm.at[tidx_vmem], rows_vmem, sem)
                pltpu.sync_copy(prior_hbm.at[i_vmem.at[0]], prior_vmem)
                c0.wait()

                @pl.loop(0, _GATHER_WINDOW)
                def _(j):
                    for k in range(ENC_DIMS // _SC_LANES):
                        base_vmem[j, k * _SC_LANES:(k + 1) * _SC_LANES] = (
                            rows_vmem[j, k * _SC_LANES:(k + 1) * _SC_LANES]
                        )

            pl.run_scoped(
                scoped,
                pltpu.VMEM((_GATHER_WINDOW,), jnp.int32),
                pltpu.VMEM((_GATHER_WINDOW, ENC_DIMS), jnp.float32),
                pltpu.SemaphoreType.DMA,
            )

        pltpu.emit_pipeline(
            body,
            grid=(n // _GATHER_WINDOW,),
            in_specs=[pl.BlockSpec((1, _GATHER_WINDOW), index_map=lambda i: (i, 0))],
            out_specs=[
                pl.BlockSpec((_GATHER_WINDOW, 2 * ENC_DIMS), index_map=lambda i: (i, 0)),
                pl.BlockSpec((_GATHER_WINDOW, PRIOR_DIM), index_map=lambda i: (i, 0)),
            ],
            core_axis_name=("core", "subcore"),
            dimension_semantics=(pltpu.PARALLEL,),
        )(i_hbm, base_out_hbm, prior_out_hbm)

    return gather_kernel(packed64, prior_matrix, idx2)


_TC_BLOCK = 8192


def _tc_body(base_ref, prior_ref, w_ref, b_ref, gamma_ref, beta_ref, o_ref):
    prior = jnp.dot(prior_ref[...], w_ref[...], preferred_element_type=jnp.float32)
    h = base_ref[:, :ENC_DIMS] + prior + b_ref[...]
    mu = jnp.mean(h, axis=-1, keepdims=True)
    d = h - mu
    var = jnp.mean(d * d, axis=-1, keepdims=True)
    o_ref[...] = (gamma_ref[...] * (d * lax.rsqrt(var + 1e-5)) + beta_ref[...]).T


def _tc_fuse(base_staged, prior_g, W, b, gamma, beta):
    n = prior_g.shape[0]
    grid = (n // _TC_BLOCK,)
    out_t = pl.pallas_call(
        _tc_body,
        grid=grid,
        in_specs=[
            pl.BlockSpec((_TC_BLOCK, 2 * ENC_DIMS), lambda i: (i, 0)),
            pl.BlockSpec((_TC_BLOCK, PRIOR_DIM), lambda i: (i, 0)),
            pl.BlockSpec((PRIOR_DIM, ENC_DIMS), lambda i: (0, 0)),
            pl.BlockSpec((1, ENC_DIMS), lambda i: (0, 0)),
            pl.BlockSpec((1, ENC_DIMS), lambda i: (0, 0)),
            pl.BlockSpec((1, ENC_DIMS), lambda i: (0, 0)),
        ],
        out_specs=pl.BlockSpec((ENC_DIMS, _TC_BLOCK), lambda i: (0, i)),
        out_shape=jax.ShapeDtypeStruct((ENC_DIMS, n), jnp.float32),
    )(base_staged, prior_g, W, b.reshape(1, -1), gamma.reshape(1, -1), beta.reshape(1, -1))
    return out_t.T


def kernel(base_emb, prior_matrix, W, b, gamma, beta, idx):
    idx2 = idx.reshape(BATCH // _GATHER_WINDOW, _GATHER_WINDOW)
    packed = _pack_base(base_emb)
    packed64 = packed.reshape(2 * PACK_ROWS, ENC_DIMS)
    base_staged, prior_g = _sc_gather(packed64, prior_matrix, idx2)
    return _tc_fuse(base_staged, prior_g, W, b, gamma, beta)


# speedup vs baseline: 1.3121x; 1.1751x over previous
import functools

import jax
import jax.numpy as jnp
from jax import lax
from jax.experimental import pallas as pl
from jax.experimental.pallas import tpu as pltpu
from jax.experimental.pallas import tpu_sc as plsc

VOCAB = 100000
ENC_DIMS = 64
PACK_ROWS = 51200
PRIOR_DIM = 128
BATCH = 16384


_PACK_COLS = 10240


def _pack_body(left_ref, right_ref, o_ref):
    both = jnp.concatenate([left_ref[...], right_ref[...]], axis=0)
    o_ref[...] = both.T


def _pack_base(base_emb):
    bt = base_emb.T
    grid = (PACK_ROWS // _PACK_COLS,)
    shift = PACK_ROWS // _PACK_COLS
    return pl.pallas_call(
        _pack_body,
        grid=grid,
        in_specs=[
            pl.BlockSpec((ENC_DIMS, _PACK_COLS), lambda i: (0, i)),
            pl.BlockSpec(
                (ENC_DIMS, _PACK_COLS),
                lambda i: (0, jnp.minimum(i + shift, (VOCAB - 1) // _PACK_COLS)),
            ),
        ],
        out_specs=pl.BlockSpec((_PACK_COLS, 2 * ENC_DIMS), lambda i: (i, 0)),
        out_shape=jax.ShapeDtypeStruct((PACK_ROWS, 2 * ENC_DIMS), jnp.float32),
    )(bt, bt)


_GATHER_WINDOW = 128
_SC_LANES = 16


def _sc_gather(packed64, prior_matrix, idx3):
    n = idx3.shape[0] * idx3.shape[1]
    nw = idx3.shape[0]
    mesh = plsc.VectorSubcoreMesh(core_axis_name="core", subcore_axis_name="subcore")
    WPT = 4

    scratch = [
        pltpu.VMEM((WPT, _GATHER_WINDOW), jnp.int32),
        pltpu.VMEM((WPT, _GATHER_WINDOW), jnp.int32),
    ]
    scratch += [pltpu.VMEM((_GATHER_WINDOW, PRIOR_DIM), jnp.float32) for _ in range(WPT)]
    scratch += [pltpu.VMEM((_GATHER_WINDOW, ENC_DIMS), jnp.float32) for _ in range(WPT)]
    scratch += [pltpu.SemaphoreType.DMA for _ in range(4 * WPT)]

    @functools.partial(
        pl.kernel,
        out_type=(
            jax.ShapeDtypeStruct((n, 2 * ENC_DIMS), jnp.float32),
            jax.ShapeDtypeStruct((n, PRIOR_DIM), jnp.float32),
        ),
        mesh=mesh,
        scratch_types=scratch,
        compiler_params=pltpu.CompilerParams(use_tc_tiling_on_sc=False),
    )
    def gather_kernel(base_hbm, prior_hbm, i_hbm, base_out_hbm, prior_out_hbm,
                      idx_v, tidx_v, *bufs_and_sems):
        pbuf = bufs_and_sems[:WPT]
        bbuf = bufs_and_sems[WPT:2 * WPT]
        sems = bufs_and_sems[2 * WPT:]
        gp_sem, gb_sem = sems[:WPT], sems[WPT:2 * WPT]
        op_sem, ob_sem = sems[2 * WPT:3 * WPT], sems[3 * WPT:]

        wid = lax.axis_index("core") * 16 + lax.axis_index("subcore")
        w0 = wid * WPT

        pltpu.sync_copy(i_hbm.at[pl.ds(w0, WPT)], idx_v)

        for j in range(WPT):
            @pl.loop(0, _GATHER_WINDOW, step=_SC_LANES)
            def _(k, j=j):
                v = idx_v[j, pl.ds(k, _SC_LANES)]
                hi = jnp.where(v >= PACK_ROWS, 1, 0)
                tidx_v[j, pl.ds(k, _SC_LANES)] = 2 * (v - PACK_ROWS * hi) + hi

        gp = [pltpu.async_copy(prior_hbm.at[idx_v.at[j]], pbuf[j], gp_sem[j])
              for j in range(WPT)]
        gb = [pltpu.async_copy(base_hbm.at[tidx_v.at[j]], bbuf[j], gb_sem[j])
              for j in range(WPT)]

        op, ob = [], []
        for j in range(WPT):
            row = (w0 + j) * _GATHER_WINDOW
            gp[j].wait()
            op.append(pltpu.async_copy(
                pbuf[j], prior_out_hbm.at[pl.ds(row, _GATHER_WINDOW)], op_sem[j]))
            gb[j].wait()
            ob.append(pltpu.async_copy(
                bbuf[j],
                base_out_hbm.at[pl.ds(row, _GATHER_WINDOW), pl.ds(0, ENC_DIMS)],
                ob_sem[j]))
        for j in range(WPT):
            op[j].wait()
            ob[j].wait()

    return gather_kernel(packed64, prior_matrix, idx3)


_TC_BLOCK = 4096


def _tc_body(base_ref, prior_ref, w_ref, b_ref, gamma_ref, beta_ref, o_ref):
    prior = jnp.dot(prior_ref[...], w_ref[...], preferred_element_type=jnp.float32)
    h = base_ref[:, :ENC_DIMS] + prior + b_ref[...]
    mu = jnp.mean(h, axis=-1, keepdims=True)
    d = h - mu
    var = jnp.mean(d * d, axis=-1, keepdims=True)
    o_ref[...] = (gamma_ref[...] * (d * lax.rsqrt(var + 1e-5)) + beta_ref[...]).T


def _tc_fuse(base_staged, prior_g, W, b, gamma, beta):
    n = prior_g.shape[0]
    grid = (n // _TC_BLOCK,)
    out_t = pl.pallas_call(
        _tc_body,
        grid=grid,
        in_specs=[
            pl.BlockSpec((_TC_BLOCK, 2 * ENC_DIMS), lambda i: (i, 0)),
            pl.BlockSpec((_TC_BLOCK, PRIOR_DIM), lambda i: (i, 0)),
            pl.BlockSpec((PRIOR_DIM, ENC_DIMS), lambda i: (0, 0)),
            pl.BlockSpec((1, ENC_DIMS), lambda i: (0, 0)),
            pl.BlockSpec((1, ENC_DIMS), lambda i: (0, 0)),
            pl.BlockSpec((1, ENC_DIMS), lambda i: (0, 0)),
        ],
        out_specs=pl.BlockSpec((ENC_DIMS, _TC_BLOCK), lambda i: (0, i)),
        out_shape=jax.ShapeDtypeStruct((ENC_DIMS, n), jnp.float32),
    )(base_staged, prior_g, W, b.reshape(1, -1), gamma.reshape(1, -1), beta.reshape(1, -1))
    return out_t.T


def kernel(base_emb, prior_matrix, W, b, gamma, beta, idx):
    idx3 = idx.reshape(BATCH // _GATHER_WINDOW, _GATHER_WINDOW)
    packed = _pack_base(base_emb)
    packed64 = packed.reshape(2 * PACK_ROWS, ENC_DIMS)
    base_staged, prior_g = _sc_gather(packed64, prior_matrix, idx3)
    return _tc_fuse(base_staged, prior_g, W, b, gamma, beta)
